# baseline (speedup 1.0000x reference)
import functools

import jax
import jax.numpy as jnp
from jax import lax
from jax.experimental import pallas as pl
from jax.experimental.pallas import tpu as pltpu
from jax.experimental.pallas import tpu_sc as plsc

_N = 10000
_E = 320000
_DIN = 128
_DOUT = 32
_NC = 2
_NS = 16
_NW = _NC * _NS
_EPT = _E // _NW
_G = 100
_NG = _EPT // _G
_RPT = _N // _NS
_NP = 10240
_NBUF = 10
_RPTP = _NP // _NS

_mesh = plsc.VectorSubcoreMesh(core_axis_name="c", subcore_axis_name="s")


@functools.partial(
    pl.kernel,
    out_type=jax.ShapeDtypeStruct((2, _NW, _NP), jnp.float32),
    scratch_types=[
        pltpu.VMEM((_NG, _G), jnp.int32),
        pltpu.VMEM((_NG, _G), jnp.int32),
        pltpu.VMEM((2 * _NP,), jnp.float32),
    ],
    mesh=_mesh,
    compiler_params=pltpu.CompilerParams(
        needs_layout_passes=False, use_tc_tiling_on_sc=False),
)
def _deg_kernel(e3_hbm, hist_hbm, idx_v, idx2_v, hist_v):
    c = lax.axis_index("c")
    s = lax.axis_index("s")
    wid = s * _NC + c
    z16 = jnp.zeros((16,), jnp.float32)
    ones16 = jnp.ones((16,), jnp.float32)

    def zbody(i, _):
        hist_v[pl.ds(i * 16, 16)] = z16
        return 0

    lax.fori_loop(0, (2 * _NP) // 16, zbody, 0)

    pltpu.sync_copy(e3_hbm.at[0, pl.ds(wid * _NG, _NG)], idx_v)
    pltpu.sync_copy(e3_hbm.at[1, pl.ds(wid * _NG, _NG)], idx2_v)

    def sbody(r, _):
        for k in range(_G // 16):
            idx = idx_v[r, pl.ds(k * 16, 16)]
            plsc.addupdate_scatter(hist_v, [idx], ones16)
        return 0

    lax.fori_loop(0, _NG, sbody, 0)

    def dbody(r, _):
        for k in range(_G // 16):
            idx = idx2_v[r, pl.ds(k * 16, 16)] + _NP
            plsc.addupdate_scatter(hist_v, [idx], ones16)
        return 0

    lax.fori_loop(0, _NG, dbody, 0)

    pltpu.sync_copy(hist_v.at[pl.ds(0, _NP)], hist_hbm.at[0, wid])
    pltpu.sync_copy(hist_v.at[pl.ds(_NP, _NP)], hist_hbm.at[1, wid])


_BLK = 1024


def _tc_matmul_body(hout_ref, hin_ref, x_ref, w_ref, y4_ref, rin4_ref):
    outdeg = jnp.sum(hout_ref[0], axis=0).reshape(_BLK)
    rout = lax.rsqrt(jnp.maximum(outdeg, 1.0))
    indeg = jnp.sum(hin_ref[0], axis=0).reshape(_BLK)
    rin = lax.rsqrt(jnp.maximum(indeg, 1.0))
    w = w_ref[0]
    w4 = jnp.concatenate([w, w, w, w], axis=1)
    z = jnp.dot(x_ref[...] * rout[:, None], w4,
                preferred_element_type=jnp.float32)
    lane = lax.broadcasted_iota(jnp.int32, (4, 128), 1)
    grp = lax.broadcasted_iota(jnp.int32, (4, 128), 0)
    m = (lane // _DOUT == grp).astype(jnp.float32)
    y4_ref[...] = (z.reshape(_BLK // 4, 4, 128) * m[None]).sum(axis=1)
    r2 = jnp.broadcast_to(rin[:, None], (_BLK, 128))
    rin4_ref[...] = (r2.reshape(_BLK // 4, 4, 128) * m[None]).sum(axis=1)


_tc_matmul = pl.pallas_call(
    _tc_matmul_body,
    grid=(10,),
    in_specs=[
        pl.BlockSpec((1, _NW, _BLK // 128, 128), lambda i: (0, 0, i, 0)),
        pl.BlockSpec((1, _NW, _BLK // 128, 128), lambda i: (1, 0, i, 0)),
        pl.BlockSpec((_BLK, _DIN), lambda i: (i, 0)),
        pl.BlockSpec((1, _DIN, _DOUT), lambda i: (0, 0, 0)),
    ],
    out_specs=[
        pl.BlockSpec((_BLK // 4, 128), lambda i: (i, 0)),
        pl.BlockSpec((_BLK // 4, 128), lambda i: (i, 0)),
    ],
    out_shape=[
        jax.ShapeDtypeStruct((_NP // 4, 128), jnp.float32),
        jax.ShapeDtypeStruct((_NP // 4, 128), jnp.float32),
    ],
)


@functools.partial(
    pl.kernel,
    out_type=jax.ShapeDtypeStruct((_NC, _NP // 4, 128), jnp.float32),
    scratch_types=[
        pltpu.VMEM((_NG, _G), jnp.int32),
        pltpu.VMEM((_NG, _G), jnp.int32),
        pltpu.VMEM((_NBUF, _G, _DOUT), jnp.float32),
        pltpu.VMEM((_RPTP, _DOUT), jnp.float32),
        pltpu.VMEM((_RPTP // 4, 128), jnp.float32),
        pltpu.VMEM_SHARED((_NP, _DOUT), jnp.float32),
        pltpu.SemaphoreType.DMA((_NBUF,)),
    ],
    mesh=_mesh,
    compiler_params=pltpu.CompilerParams(use_tc_tiling_on_sc=False),
)
def _agg_kernel(e3_hbm, y_hbm, out_hbm,
                si_v, di_v, rows_v, buf_v, buf2_v, agg_sh, sem):
    c = lax.axis_index("c")
    s = lax.axis_index("s")
    wid = s * _NC + c
    z16 = jnp.zeros((16,), jnp.float32)

    def zbody(i, _):
        buf_v[i, pl.ds(0, 16)] = z16
        buf_v[i, pl.ds(16, 16)] = z16
        return 0

    lax.fori_loop(0, _RPTP, zbody, 0)
    pltpu.sync_copy(buf_v, agg_sh.at[pl.ds(s * _RPTP, _RPTP)])
    plsc.subcore_barrier()

    pltpu.sync_copy(e3_hbm.at[0, pl.ds(wid * _NG, _NG)], si_v)
    pltpu.sync_copy(e3_hbm.at[1, pl.ds(wid * _NG, _NG)], di_v)

    for b in range(_NBUF):
        pltpu.async_copy(y_hbm.at[si_v.at[b]], rows_v.at[b], sem.at[b])

    def gbody(g0, _):
        for b in range(_NBUF):
            g = g0 * _NBUF + b
            pltpu.make_async_copy(
                y_hbm.at[si_v.at[g]], rows_v.at[b], sem.at[b]
            ).wait()
            pltpu.sync_copy(rows_v.at[b], agg_sh.at[di_v.at[g]], add=True)

            @pl.when(g + _NBUF < _NG)
            def _():
                pltpu.async_copy(
                    y_hbm.at[si_v.at[g + _NBUF]], rows_v.at[b], sem.at[b]
                )

        return 0

    lax.fori_loop(0, _NG // _NBUF, gbody, 0)
    plsc.subcore_barrier()

    pltpu.sync_copy(agg_sh.at[pl.ds(s * _RPTP, _RPTP)], buf_v)

    def rbody(r, _):
        for a in range(4):
            buf2_v[r, pl.ds(a * _DOUT, 16)] = buf_v[4 * r + a, pl.ds(0, 16)]
            buf2_v[r, pl.ds(a * _DOUT + 16, 16)] = buf_v[4 * r + a, pl.ds(16, 16)]
        return 0

    lax.fori_loop(0, _RPTP // 4, rbody, 0)
    pltpu.sync_copy(buf2_v, out_hbm.at[c, pl.ds(s * (_RPTP // 4), _RPTP // 4)])


def _tc_combine_body(p_ref, rin4_ref, o_ref):
    o_ref[...] = (p_ref[0] + p_ref[1]) * rin4_ref[...]


_tc_combine = pl.pallas_call(
    _tc_combine_body,
    grid=(10,),
    in_specs=[
        pl.BlockSpec((_NC, _BLK // 4, 128), lambda i: (0, i, 0)),
        pl.BlockSpec((_BLK // 4, 128), lambda i: (i, 0)),
    ],
    out_specs=pl.BlockSpec((_BLK // 4, 128), lambda i: (i, 0)),
    out_shape=jax.ShapeDtypeStruct((_N // 4, 128), jnp.float32),
)


def kernel(edge_index, x, W):
    e3 = edge_index.reshape(2, _NW * _NG, _G)
    hist = _deg_kernel(e3).reshape(2, _NW, _NP // 128, 128)
    y4, rin4 = _tc_matmul(hist, hist, x, W)
    p4 = _agg_kernel(e3, y4.reshape(_NP, _DOUT))
    return _tc_combine(p4, rin4).reshape(_N, _DOUT)


# speedup vs baseline: 1.0756x; 1.0756x over previous
import functools

import jax
import jax.numpy as jnp
from jax import lax
from jax.experimental import pallas as pl
from jax.experimental.pallas import tpu as pltpu
from jax.experimental.pallas import tpu_sc as plsc

_N = 10000
_E = 320000
_DIN = 128
_DOUT = 32
_NC = 2
_NS = 16
_NW = _NC * _NS
_EPT = _E // _NW
_G = 80
_NG = _EPT // _G
_RPT = _N // _NS
_NP = 10240
_NBUF = 5
_RPTP = _NP // _NS

_mesh = plsc.VectorSubcoreMesh(core_axis_name="c", subcore_axis_name="s")


@functools.partial(
    pl.kernel,
    out_type=jax.ShapeDtypeStruct((2, _NW, _NP), jnp.float32),
    scratch_types=[
        pltpu.VMEM((_NG, _G), jnp.int32),
        pltpu.VMEM((_NG, _G), jnp.int32),
        pltpu.VMEM((2 * _NP,), jnp.float32),
    ],
    mesh=_mesh,
    compiler_params=pltpu.CompilerParams(
        needs_layout_passes=False, use_tc_tiling_on_sc=False),
)
def _deg_kernel(e3_hbm, hist_hbm, idx_v, idx2_v, hist_v):
    c = lax.axis_index("c")
    s = lax.axis_index("s")
    wid = s * _NC + c
    z16 = jnp.zeros((16,), jnp.float32)
    ones16 = jnp.ones((16,), jnp.float32)

    def zbody(i, _):
        hist_v[pl.ds(i * 16, 16)] = z16
        return 0

    lax.fori_loop(0, (2 * _NP) // 16, zbody, 0)

    pltpu.sync_copy(e3_hbm.at[0, pl.ds(wid * _NG, _NG)], idx_v)
    pltpu.sync_copy(e3_hbm.at[1, pl.ds(wid * _NG, _NG)], idx2_v)

    def sbody(r, _):
        for k in range(_G // 16):
            idx = idx_v[r, pl.ds(k * 16, 16)]
            plsc.addupdate_scatter(hist_v, [idx], ones16)
        return 0

    lax.fori_loop(0, _NG, sbody, 0)

    def dbody(r, _):
        for k in range(_G // 16):
            idx = idx2_v[r, pl.ds(k * 16, 16)] + _NP
            plsc.addupdate_scatter(hist_v, [idx], ones16)
        return 0

    lax.fori_loop(0, _NG, dbody, 0)

    pltpu.sync_copy(hist_v.at[pl.ds(0, _NP)], hist_hbm.at[0, wid])
    pltpu.sync_copy(hist_v.at[pl.ds(_NP, _NP)], hist_hbm.at[1, wid])


_BLK = 1024


def _tc_matmul_body(x_ref, w_ref, y4_ref):
    w = w_ref[0]
    w4 = jnp.concatenate([w, w, w, w], axis=1)
    z = jnp.dot(x_ref[...], w4, preferred_element_type=jnp.float32)
    lane = lax.broadcasted_iota(jnp.int32, (4, 128), 1)
    grp = lax.broadcasted_iota(jnp.int32, (4, 128), 0)
    m = (lane // _DOUT == grp).astype(jnp.float32)
    y4_ref[...] = (z.reshape(_BLK // 4, 4, 128) * m[None]).sum(axis=1)


_tc_matmul = pl.pallas_call(
    _tc_matmul_body,
    grid=(10,),
    in_specs=[
        pl.BlockSpec((_BLK, _DIN), lambda i: (i, 0)),
        pl.BlockSpec((1, _DIN, _DOUT), lambda i: (0, 0, 0)),
    ],
    out_specs=pl.BlockSpec((_BLK // 4, 128), lambda i: (i, 0)),
    out_shape=jax.ShapeDtypeStruct((_NP // 4, 128), jnp.float32),
)


def _tc_scale_body(hout_ref, hin_ref, y4raw_ref, y4_ref, rin4_ref):
    outdeg = jnp.sum(hout_ref[0], axis=0).reshape(_BLK)
    rout = lax.rsqrt(jnp.maximum(outdeg, 1.0))
    indeg = jnp.sum(hin_ref[0], axis=0).reshape(_BLK)
    rin = lax.rsqrt(jnp.maximum(indeg, 1.0))
    lane = lax.broadcasted_iota(jnp.int32, (4, 128), 1)
    grp = lax.broadcasted_iota(jnp.int32, (4, 128), 0)
    m = (lane // _DOUT == grp).astype(jnp.float32)[None]
    ro2 = jnp.broadcast_to(rout[:, None], (_BLK, 128))
    rout4 = (ro2.reshape(_BLK // 4, 4, 128) * m).sum(axis=1)
    y4_ref[...] = y4raw_ref[...] * rout4
    ri2 = jnp.broadcast_to(rin[:, None], (_BLK, 128))
    rin4_ref[...] = (ri2.reshape(_BLK // 4, 4, 128) * m).sum(axis=1)


_tc_scale = pl.pallas_call(
    _tc_scale_body,
    grid=(10,),
    in_specs=[
        pl.BlockSpec((1, _NW, _BLK // 128, 128), lambda i: (0, 0, i, 0)),
        pl.BlockSpec((1, _NW, _BLK // 128, 128), lambda i: (1, 0, i, 0)),
        pl.BlockSpec((_BLK // 4, 128), lambda i: (i, 0)),
    ],
    out_specs=[
        pl.BlockSpec((_BLK // 4, 128), lambda i: (i, 0)),
        pl.BlockSpec((_BLK // 4, 128), lambda i: (i, 0)),
    ],
    out_shape=[
        jax.ShapeDtypeStruct((_NP // 4, 128), jnp.float32),
        jax.ShapeDtypeStruct((_NP // 4, 128), jnp.float32),
    ],
)


@functools.partial(
    pl.kernel,
    out_type=jax.ShapeDtypeStruct((_NC, _NP // 4, 128), jnp.float32),
    scratch_types=[
        pltpu.VMEM((_NG, _G), jnp.int32),
        pltpu.VMEM((_NG, _G), jnp.int32),
        pltpu.VMEM((_NBUF, _G, _DOUT), jnp.float32),
        pltpu.VMEM((_RPTP, _DOUT), jnp.float32),
        pltpu.VMEM((_RPTP // 4, 128), jnp.float32),
        pltpu.VMEM_SHARED((_NP, _DOUT), jnp.float32),
        pltpu.SemaphoreType.DMA((_NBUF,)),
    ],
    mesh=_mesh,
    compiler_params=pltpu.CompilerParams(use_tc_tiling_on_sc=False),
)
def _agg_kernel(e3_hbm, y_hbm, out_hbm,
                si_v, di_v, rows_v, buf_v, buf2_v, agg_sh, sem):
    c = lax.axis_index("c")
    s = lax.axis_index("s")
    wid = s * _NC + c
    z16 = jnp.zeros((16,), jnp.float32)

    def zbody(i, _):
        buf_v[i, pl.ds(0, 16)] = z16
        buf_v[i, pl.ds(16, 16)] = z16
        return 0

    lax.fori_loop(0, _RPTP, zbody, 0)
    pltpu.sync_copy(buf_v, agg_sh.at[pl.ds(s * _RPTP, _RPTP)])
    plsc.subcore_barrier()

    pltpu.sync_copy(e3_hbm.at[0, pl.ds(wid * _NG, _NG)], si_v)
    pltpu.sync_copy(e3_hbm.at[1, pl.ds(wid * _NG, _NG)], di_v)

    for b in range(_NBUF):
        pltpu.async_copy(y_hbm.at[si_v.at[b]], rows_v.at[b], sem.at[b])

    def gbody(g0, _):
        for b in range(_NBUF):
            g = g0 * _NBUF + b
            pltpu.make_async_copy(
                y_hbm.at[si_v.at[g]], rows_v.at[b], sem.at[b]
            ).wait()
            pltpu.sync_copy(rows_v.at[b], agg_sh.at[di_v.at[g]], add=True)

            @pl.when(g + _NBUF < _NG)
            def _():
                pltpu.async_copy(
                    y_hbm.at[si_v.at[g + _NBUF]], rows_v.at[b], sem.at[b]
                )

        return 0

    lax.fori_loop(0, _NG // _NBUF, gbody, 0)
    plsc.subcore_barrier()

    pltpu.sync_copy(agg_sh.at[pl.ds(s * _RPTP, _RPTP)], buf_v)

    def rbody(r, _):
        for a in range(4):
            buf2_v[r, pl.ds(a * _DOUT, 16)] = buf_v[4 * r + a, pl.ds(0, 16)]
            buf2_v[r, pl.ds(a * _DOUT + 16, 16)] = buf_v[4 * r + a, pl.ds(16, 16)]
        return 0

    lax.fori_loop(0, _RPTP // 4, rbody, 0)
    pltpu.sync_copy(buf2_v, out_hbm.at[c, pl.ds(s * (_RPTP // 4), _RPTP // 4)])


def _tc_combine_body(p_ref, rin4_ref, o_ref):
    o_ref[...] = (p_ref[0] + p_ref[1]) * rin4_ref[...]


_tc_combine = pl.pallas_call(
    _tc_combine_body,
    grid=(10,),
    in_specs=[
        pl.BlockSpec((_NC, _BLK // 4, 128), lambda i: (0, i, 0)),
        pl.BlockSpec((_BLK // 4, 128), lambda i: (i, 0)),
    ],
    out_specs=pl.BlockSpec((_BLK // 4, 128), lambda i: (i, 0)),
    out_shape=jax.ShapeDtypeStruct((_N // 4, 128), jnp.float32),
)


def kernel(edge_index, x, W):
    e3 = edge_index.reshape(2, _NW * _NG, _G)
    hist = _deg_kernel(e3).reshape(2, _NW, _NP // 128, 128)
    y4raw = _tc_matmul(x, W)
    y4, rin4 = _tc_scale(hist, hist, y4raw)
    p4 = _agg_kernel(e3, y4.reshape(_NP, _DOUT))
    return _tc_combine(p4, rin4).reshape(_N, _DOUT)


# speedup vs baseline: 1.1042x; 1.0266x over previous
import functools

import jax
import jax.numpy as jnp
from jax import lax
from jax.experimental import pallas as pl
from jax.experimental.pallas import tpu as pltpu
from jax.experimental.pallas import tpu_sc as plsc

_N = 10000
_E = 320000
_DIN = 128
_DOUT = 32
_NC = 2
_NS = 16
_NW = _NC * _NS
_EPT = _E // _NW
_G = 80
_NG = _EPT // _G
_RPT = _N // _NS
_NP = 10240
_NSLOT = 10
_KAHEAD = 6
_RPTP = _NP // _NS

_mesh = plsc.VectorSubcoreMesh(core_axis_name="c", subcore_axis_name="s")


@functools.partial(
    pl.kernel,
    out_type=jax.ShapeDtypeStruct((2, _NW, _NP), jnp.float32),
    scratch_types=[
        pltpu.VMEM((_NG, _G), jnp.int32),
        pltpu.VMEM((_NG, _G), jnp.int32),
        pltpu.VMEM((2 * _NP,), jnp.float32),
    ],
    mesh=_mesh,
    compiler_params=pltpu.CompilerParams(
        needs_layout_passes=False, use_tc_tiling_on_sc=False),
)
def _deg_kernel(e3_hbm, hist_hbm, idx_v, idx2_v, hist_v):
    c = lax.axis_index("c")
    s = lax.axis_index("s")
    wid = s * _NC + c
    z16 = jnp.zeros((16,), jnp.float32)
    ones16 = jnp.ones((16,), jnp.float32)

    def zbody(i, _):
        hist_v[pl.ds(i * 16, 16)] = z16
        return 0

    lax.fori_loop(0, (2 * _NP) // 16, zbody, 0)

    pltpu.sync_copy(e3_hbm.at[0, pl.ds(wid * _NG, _NG)], idx_v)
    pltpu.sync_copy(e3_hbm.at[1, pl.ds(wid * _NG, _NG)], idx2_v)

    def sbody(r, _):
        for k in range(_G // 16):
            idx = idx_v[r, pl.ds(k * 16, 16)]
            plsc.addupdate_scatter(hist_v, [idx], ones16)
        return 0

    lax.fori_loop(0, _NG, sbody, 0)

    def dbody(r, _):
        for k in range(_G // 16):
            idx = idx2_v[r, pl.ds(k * 16, 16)] + _NP
            plsc.addupdate_scatter(hist_v, [idx], ones16)
        return 0

    lax.fori_loop(0, _NG, dbody, 0)

    pltpu.sync_copy(hist_v.at[pl.ds(0, _NP)], hist_hbm.at[0, wid])
    pltpu.sync_copy(hist_v.at[pl.ds(_NP, _NP)], hist_hbm.at[1, wid])


_BLK = 1024


def _tc_matmul_body(x_ref, w_ref, y4_ref):
    w = w_ref[0]
    w4 = jnp.concatenate([w, w, w, w], axis=1)
    z = jnp.dot(x_ref[...], w4, preferred_element_type=jnp.float32)
    lane = lax.broadcasted_iota(jnp.int32, (4, 128), 1)
    grp = lax.broadcasted_iota(jnp.int32, (4, 128), 0)
    m = (lane // _DOUT == grp).astype(jnp.float32)
    y4_ref[...] = (z.reshape(_BLK // 4, 4, 128) * m[None]).sum(axis=1)


_tc_matmul = pl.pallas_call(
    _tc_matmul_body,
    grid=(10,),
    in_specs=[
        pl.BlockSpec((_BLK, _DIN), lambda i: (i, 0)),
        pl.BlockSpec((1, _DIN, _DOUT), lambda i: (0, 0, 0)),
    ],
    out_specs=pl.BlockSpec((_BLK // 4, 128), lambda i: (i, 0)),
    out_shape=jax.ShapeDtypeStruct((_NP // 4, 128), jnp.float32),
)


def _tc_scale_body(hout_ref, hin_ref, y4raw_ref, y4_ref, rin4_ref):
    outdeg = jnp.sum(hout_ref[0], axis=0).reshape(_BLK)
    rout = lax.rsqrt(jnp.maximum(outdeg, 1.0))
    indeg = jnp.sum(hin_ref[0], axis=0).reshape(_BLK)
    rin = lax.rsqrt(jnp.maximum(indeg, 1.0))
    lane = lax.broadcasted_iota(jnp.int32, (4, 128), 1)
    grp = lax.broadcasted_iota(jnp.int32, (4, 128), 0)
    m = (lane // _DOUT == grp).astype(jnp.float32)[None]
    ro2 = jnp.broadcast_to(rout[:, None], (_BLK, 128))
    rout4 = (ro2.reshape(_BLK // 4, 4, 128) * m).sum(axis=1)
    y4_ref[...] = y4raw_ref[...] * rout4
    ri2 = jnp.broadcast_to(rin[:, None], (_BLK, 128))
    rin4_ref[...] = (ri2.reshape(_BLK // 4, 4, 128) * m).sum(axis=1)


_tc_scale = pl.pallas_call(
    _tc_scale_body,
    grid=(10,),
    in_specs=[
        pl.BlockSpec((1, _NW, _BLK // 128, 128), lambda i: (0, 0, i, 0)),
        pl.BlockSpec((1, _NW, _BLK // 128, 128), lambda i: (1, 0, i, 0)),
        pl.BlockSpec((_BLK // 4, 128), lambda i: (i, 0)),
    ],
    out_specs=[
        pl.BlockSpec((_BLK // 4, 128), lambda i: (i, 0)),
        pl.BlockSpec((_BLK // 4, 128), lambda i: (i, 0)),
    ],
    out_shape=[
        jax.ShapeDtypeStruct((_NP // 4, 128), jnp.float32),
        jax.ShapeDtypeStruct((_NP // 4, 128), jnp.float32),
    ],
)


@functools.partial(
    pl.kernel,
    out_type=jax.ShapeDtypeStruct((_NC, _NP // 4, 128), jnp.float32),
    scratch_types=[
        pltpu.VMEM((_NG, _G), jnp.int32),
        pltpu.VMEM((_NG, _G), jnp.int32),
        pltpu.VMEM((_NSLOT, _G, _DOUT), jnp.float32),
        pltpu.VMEM((_RPTP, _DOUT), jnp.float32),
        pltpu.VMEM((_RPTP // 4, 128), jnp.float32),
        pltpu.VMEM_SHARED((_NP, _DOUT), jnp.float32),
        pltpu.SemaphoreType.DMA((_NSLOT,)),
        pltpu.SemaphoreType.DMA((_NSLOT,)),
    ],
    mesh=_mesh,
    compiler_params=pltpu.CompilerParams(use_tc_tiling_on_sc=False),
)
def _agg_kernel(e3_hbm, y_hbm, out_hbm,
                si_v, di_v, rows_v, buf_v, buf2_v, agg_sh, gsem, ssem):
    c = lax.axis_index("c")
    s = lax.axis_index("s")
    wid = s * _NC + c
    z16 = jnp.zeros((16,), jnp.float32)

    def zbody(i, _):
        buf_v[i, pl.ds(0, 16)] = z16
        buf_v[i, pl.ds(16, 16)] = z16
        return 0

    lax.fori_loop(0, _RPTP, zbody, 0)
    pltpu.sync_copy(buf_v, agg_sh.at[pl.ds(s * _RPTP, _RPTP)])
    plsc.subcore_barrier()

    pltpu.sync_copy(e3_hbm.at[0, pl.ds(wid * _NG, _NG)], si_v)
    pltpu.sync_copy(e3_hbm.at[1, pl.ds(wid * _NG, _NG)], di_v)

    for b in range(_KAHEAD):
        pltpu.async_copy(y_hbm.at[si_v.at[b]], rows_v.at[b], gsem.at[b])

    def visit(v, b):
        pltpu.make_async_copy(
            y_hbm.at[si_v.at[v]], rows_v.at[b], gsem.at[b]
        ).wait()
        pltpu.async_copy(
            rows_v.at[b], agg_sh.at[di_v.at[v]], ssem.at[b], add=True
        )
        gn = v + _KAHEAD
        bs = (b + _KAHEAD) % _NSLOT

        @pl.when(gn < _NG)
        def _():
            @pl.when(v >= _NSLOT - _KAHEAD)
            def _():
                pltpu.make_async_copy(
                    rows_v.at[bs], agg_sh.at[di_v.at[v]], ssem.at[bs]
                ).wait()

            pltpu.async_copy(y_hbm.at[si_v.at[gn]], rows_v.at[bs], gsem.at[bs])

    def gbody(g0, _):
        for k in range(_NSLOT):
            visit(g0 * _NSLOT + k, k)
        return 0

    lax.fori_loop(0, (_NG // _NSLOT) * _NSLOT // _NSLOT, gbody, 0)

    for k in range(_NG % _NSLOT):
        visit((_NG // _NSLOT) * _NSLOT + k, k)

    for b in range(_NSLOT):
        pltpu.make_async_copy(
            rows_v.at[b], agg_sh.at[di_v.at[b]], ssem.at[b]
        ).wait()
    plsc.subcore_barrier()

    pltpu.sync_copy(agg_sh.at[pl.ds(s * _RPTP, _RPTP)], buf_v)

    def rbody(r, _):
        for a in range(4):
            buf2_v[r, pl.ds(a * _DOUT, 16)] = buf_v[4 * r + a, pl.ds(0, 16)]
            buf2_v[r, pl.ds(a * _DOUT + 16, 16)] = buf_v[4 * r + a, pl.ds(16, 16)]
        return 0

    lax.fori_loop(0, _RPTP // 4, rbody, 0)
    pltpu.sync_copy(buf2_v, out_hbm.at[c, pl.ds(s * (_RPTP // 4), _RPTP // 4)])


def _tc_combine_body(p_ref, rin4_ref, o_ref):
    o_ref[...] = (p_ref[0] + p_ref[1]) * rin4_ref[...]


_tc_combine = pl.pallas_call(
    _tc_combine_body,
    grid=(10,),
    in_specs=[
        pl.BlockSpec((_NC, _BLK // 4, 128), lambda i: (0, i, 0)),
        pl.BlockSpec((_BLK // 4, 128), lambda i: (i, 0)),
    ],
    out_specs=pl.BlockSpec((_BLK // 4, 128), lambda i: (i, 0)),
    out_shape=jax.ShapeDtypeStruct((_N // 4, 128), jnp.float32),
)


def kernel(edge_index, x, W):
    e3 = edge_index.reshape(2, _NW * _NG, _G)
    hist = _deg_kernel(e3).reshape(2, _NW, _NP // 128, 128)
    y4raw = _tc_matmul(x, W)
    y4, rin4 = _tc_scale(hist, hist, y4raw)
    p4 = _agg_kernel(e3, y4.reshape(_NP, _DOUT))
    return _tc_combine(p4, rin4).reshape(_N, _DOUT)


# speedup vs baseline: 1.1209x; 1.0151x over previous
import functools

import jax
import jax.numpy as jnp
from jax import lax
from jax.experimental import pallas as pl
from jax.experimental.pallas import tpu as pltpu
from jax.experimental.pallas import tpu_sc as plsc

_N = 10000
_E = 320000
_DIN = 128
_DOUT = 32
_NC = 2
_NS = 16
_NW = _NC * _NS
_EPT = _E // _NW
_G = 80
_NG = _EPT // _G
_RPT = _N // _NS
_NP = 10240
_NSLOT = 15
_KAHEAD = 9
_RPTP = _NP // _NS

_mesh = plsc.VectorSubcoreMesh(core_axis_name="c", subcore_axis_name="s")


@functools.partial(
    pl.kernel,
    out_type=jax.ShapeDtypeStruct((2, _NW, _NP), jnp.float32),
    scratch_types=[
        pltpu.VMEM((_NG, _G), jnp.int32),
        pltpu.VMEM((_NG, _G), jnp.int32),
        pltpu.VMEM((2 * _NP,), jnp.float32),
    ],
    mesh=_mesh,
    compiler_params=pltpu.CompilerParams(
        needs_layout_passes=False, use_tc_tiling_on_sc=False),
)
def _deg_kernel(e3_hbm, hist_hbm, idx_v, idx2_v, hist_v):
    c = lax.axis_index("c")
    s = lax.axis_index("s")
    wid = s * _NC + c
    z16 = jnp.zeros((16,), jnp.float32)
    ones16 = jnp.ones((16,), jnp.float32)

    def zbody(i, _):
        hist_v[pl.ds(i * 16, 16)] = z16
        return 0

    lax.fori_loop(0, (2 * _NP) // 16, zbody, 0)

    pltpu.sync_copy(e3_hbm.at[0, pl.ds(wid * _NG, _NG)], idx_v)
    pltpu.sync_copy(e3_hbm.at[1, pl.ds(wid * _NG, _NG)], idx2_v)

    def sbody(r, _):
        for k in range(_G // 16):
            idx = idx_v[r, pl.ds(k * 16, 16)]
            plsc.addupdate_scatter(hist_v, [idx], ones16)
        return 0

    lax.fori_loop(0, _NG, sbody, 0)

    def dbody(r, _):
        for k in range(_G // 16):
            idx = idx2_v[r, pl.ds(k * 16, 16)] + _NP
            plsc.addupdate_scatter(hist_v, [idx], ones16)
        return 0

    lax.fori_loop(0, _NG, dbody, 0)

    pltpu.sync_copy(hist_v.at[pl.ds(0, _NP)], hist_hbm.at[0, wid])
    pltpu.sync_copy(hist_v.at[pl.ds(_NP, _NP)], hist_hbm.at[1, wid])


_BLK = 1024


def _tc_matmul_body(x_ref, w_ref, y4_ref):
    w = w_ref[0]
    w4 = jnp.concatenate([w, w, w, w], axis=1)
    z = jnp.dot(x_ref[...], w4, preferred_element_type=jnp.float32)
    lane = lax.broadcasted_iota(jnp.int32, (4, 128), 1)
    grp = lax.broadcasted_iota(jnp.int32, (4, 128), 0)
    m = (lane // _DOUT == grp).astype(jnp.float32)
    y4_ref[...] = (z.reshape(_BLK // 4, 4, 128) * m[None]).sum(axis=1)


_tc_matmul = pl.pallas_call(
    _tc_matmul_body,
    grid=(10,),
    in_specs=[
        pl.BlockSpec((_BLK, _DIN), lambda i: (i, 0)),
        pl.BlockSpec((1, _DIN, _DOUT), lambda i: (0, 0, 0)),
    ],
    out_specs=pl.BlockSpec((_BLK // 4, 128), lambda i: (i, 0)),
    out_shape=jax.ShapeDtypeStruct((_NP // 4, 128), jnp.float32),
)


def _tc_scale_body(hout_ref, hin_ref, y4raw_ref, y4_ref, rin4_ref):
    outdeg = jnp.sum(hout_ref[0], axis=0).reshape(_BLK)
    rout = lax.rsqrt(jnp.maximum(outdeg, 1.0))
    indeg = jnp.sum(hin_ref[0], axis=0).reshape(_BLK)
    rin = lax.rsqrt(jnp.maximum(indeg, 1.0))
    lane = lax.broadcasted_iota(jnp.int32, (4, 128), 1)
    grp = lax.broadcasted_iota(jnp.int32, (4, 128), 0)
    m = (lane // _DOUT == grp).astype(jnp.float32)[None]
    ro2 = jnp.broadcast_to(rout[:, None], (_BLK, 128))
    rout4 = (ro2.reshape(_BLK // 4, 4, 128) * m).sum(axis=1)
    y4_ref[...] = y4raw_ref[...] * rout4
    ri2 = jnp.broadcast_to(rin[:, None], (_BLK, 128))
    rin4_ref[...] = (ri2.reshape(_BLK // 4, 4, 128) * m).sum(axis=1)


_tc_scale = pl.pallas_call(
    _tc_scale_body,
    grid=(10,),
    in_specs=[
        pl.BlockSpec((1, _NW, _BLK // 128, 128), lambda i: (0, 0, i, 0)),
        pl.BlockSpec((1, _NW, _BLK // 128, 128), lambda i: (1, 0, i, 0)),
        pl.BlockSpec((_BLK // 4, 128), lambda i: (i, 0)),
    ],
    out_specs=[
        pl.BlockSpec((_BLK // 4, 128), lambda i: (i, 0)),
        pl.BlockSpec((_BLK // 4, 128), lambda i: (i, 0)),
    ],
    out_shape=[
        jax.ShapeDtypeStruct((_NP // 4, 128), jnp.float32),
        jax.ShapeDtypeStruct((_NP // 4, 128), jnp.float32),
    ],
)


@functools.partial(
    pl.kernel,
    out_type=jax.ShapeDtypeStruct((_NC, _NP // 4, 128), jnp.float32),
    scratch_types=[
        pltpu.VMEM((_NG, _G), jnp.int32),
        pltpu.VMEM((_NG, _G), jnp.int32),
        pltpu.VMEM((_NSLOT, _G, _DOUT), jnp.float32),
        pltpu.VMEM((_RPTP, _DOUT), jnp.float32),
        pltpu.VMEM((_RPTP // 4, 128), jnp.float32),
        pltpu.VMEM_SHARED((_NP, _DOUT), jnp.float32),
        pltpu.SemaphoreType.DMA((_NSLOT,)),
        pltpu.SemaphoreType.DMA((_NSLOT,)),
    ],
    mesh=_mesh,
    compiler_params=pltpu.CompilerParams(use_tc_tiling_on_sc=False),
)
def _agg_kernel(e3_hbm, y_hbm, out_hbm,
                si_v, di_v, rows_v, buf_v, buf2_v, agg_sh, gsem, ssem):
    c = lax.axis_index("c")
    s = lax.axis_index("s")
    wid = s * _NC + c
    z16 = jnp.zeros((16,), jnp.float32)

    def zbody(i, _):
        buf_v[i, pl.ds(0, 16)] = z16
        buf_v[i, pl.ds(16, 16)] = z16
        return 0

    lax.fori_loop(0, _RPTP, zbody, 0)
    pltpu.sync_copy(buf_v, agg_sh.at[pl.ds(s * _RPTP, _RPTP)])
    plsc.subcore_barrier()

    pltpu.sync_copy(e3_hbm.at[0, pl.ds(wid * _NG, _NG)], si_v)
    pltpu.sync_copy(e3_hbm.at[1, pl.ds(wid * _NG, _NG)], di_v)

    for b in range(_KAHEAD):
        pltpu.async_copy(y_hbm.at[si_v.at[b]], rows_v.at[b], gsem.at[b])

    def visit(v, b):
        pltpu.make_async_copy(
            y_hbm.at[si_v.at[v]], rows_v.at[b], gsem.at[b]
        ).wait()
        pltpu.async_copy(
            rows_v.at[b], agg_sh.at[di_v.at[v]], ssem.at[b], add=True
        )
        gn = v + _KAHEAD
        bs = (b + _KAHEAD) % _NSLOT

        @pl.when(gn < _NG)
        def _():
            @pl.when(v >= _NSLOT - _KAHEAD)
            def _():
                pltpu.make_async_copy(
                    rows_v.at[bs], agg_sh.at[di_v.at[v]], ssem.at[bs]
                ).wait()

            pltpu.async_copy(y_hbm.at[si_v.at[gn]], rows_v.at[bs], gsem.at[bs])

    def gbody(g0, _):
        for k in range(_NSLOT):
            visit(g0 * _NSLOT + k, k)
        return 0

    lax.fori_loop(0, (_NG // _NSLOT) * _NSLOT // _NSLOT, gbody, 0)

    for k in range(_NG % _NSLOT):
        visit((_NG // _NSLOT) * _NSLOT + k, k)

    for b in range(_NSLOT):
        pltpu.make_async_copy(
            rows_v.at[b], agg_sh.at[di_v.at[b]], ssem.at[b]
        ).wait()
    plsc.subcore_barrier()

    pltpu.sync_copy(agg_sh.at[pl.ds(s * _RPTP, _RPTP)], buf_v)

    def rbody(r, _):
        for a in range(4):
            buf2_v[r, pl.ds(a * _DOUT, 16)] = buf_v[4 * r + a, pl.ds(0, 16)]
            buf2_v[r, pl.ds(a * _DOUT + 16, 16)] = buf_v[4 * r + a, pl.ds(16, 16)]
        return 0

    lax.fori_loop(0, _RPTP // 4, rbody, 0)
    pltpu.sync_copy(buf2_v, out_hbm.at[c, pl.ds(s * (_RPTP // 4), _RPTP // 4)])


def _tc_combine_body(p_ref, rin4_ref, o_ref):
    o_ref[...] = (p_ref[0] + p_ref[1]) * rin4_ref[...]


_tc_combine = pl.pallas_call(
    _tc_combine_body,
    grid=(10,),
    in_specs=[
        pl.BlockSpec((_NC, _BLK // 4, 128), lambda i: (0, i, 0)),
        pl.BlockSpec((_BLK // 4, 128), lambda i: (i, 0)),
    ],
    out_specs=pl.BlockSpec((_BLK // 4, 128), lambda i: (i, 0)),
    out_shape=jax.ShapeDtypeStruct((_N // 4, 128), jnp.float32),
)


def kernel(edge_index, x, W):
    e3 = edge_index.reshape(2, _NW * _NG, _G)
    hist = _deg_kernel(e3).reshape(2, _NW, _NP // 128, 128)
    y4raw = _tc_matmul(x, W)
    y4, rin4 = _tc_scale(hist, hist, y4raw)
    p4 = _agg_kernel(e3, y4.reshape(_NP, _DOUT))
    return _tc_combine(p4, rin4).reshape(_N, _DOUT)
